# baseline (speedup 1.0000x reference)
import jax
import jax.numpy as jnp
from jax import lax
from jax.experimental import pallas as pl
from jax.experimental.pallas import tpu as pltpu
from jax.experimental.pallas import tpu_sc as plsc

B = 16
P = 512 * 512
NB = 8192
T = 8.0
NW = 32
NR = 8192
NC = 512
RW = NR // NW
RB = 32
NCH = RW // RB
CHW = RB * NC
ROW = 4 * NB
SCALE = NB / T


def _sc_hist_body(logits_hbm, labels_hbm, out_hbm, hist, lb0, lb1, gb0, gb1,
                  sl0, sg0, sl1, sg1):
    w = lax.axis_index("s") * 2 + lax.axis_index("c")
    img = w % B
    half = w // B
    r0 = img * (NR // B) + half * RW

    @plsc.parallel_loop(0, ROW, 16, unroll=8)
    def _(i):
        hist[pl.ds(i, 16)] = jnp.zeros((16,), jnp.float32)

    def start_copy(ci, lbuf, gbuf, seml, semg):
        rows = pl.ds(r0 + ci * RB, RB)
        pltpu.make_async_copy(logits_hbm.at[rows], lbuf, seml).start()
        pltpu.make_async_copy(labels_hbm.at[rows], gbuf, semg).start()

    def wait_copy(ci, lbuf, gbuf, seml, semg):
        rows = pl.ds(r0 + ci * RB, RB)
        pltpu.make_async_copy(logits_hbm.at[rows], lbuf, seml).wait()
        pltpu.make_async_copy(labels_hbm.at[rows], gbuf, semg).wait()

    def process(lbuf, gbuf):
        @plsc.parallel_loop(0, CHW, 16, unroll=8)
        def _(off):
            r = lax.shift_right_logical(off, 9)
            c = lax.bitwise_and(off, NC - 1)
            lv = lbuf[r, pl.ds(c, 16)]
            gv = gbuf[r, pl.ds(c, 16)]
            gf = gv.astype(jnp.float32)
            e = jnp.maximum(1.0 - lv * (2.0 * gf - 1.0), 0.0)
            ki = jnp.minimum((e * SCALE).astype(jnp.int32), NB - 1)
            idx = lax.shift_left(gv, 13) + ki
            plsc.addupdate_scatter(hist, [idx], jnp.ones((16,), jnp.float32))
            plsc.addupdate_scatter(hist, [idx + 2 * NB], e)

    start_copy(0, lb0, gb0, sl0, sg0)

    def pair_body(ii, carry):
        c0 = 2 * ii
        start_copy(c0 + 1, lb1, gb1, sl1, sg1)
        wait_copy(c0, lb0, gb0, sl0, sg0)
        process(lb0, gb0)

        @pl.when(ii < NCH // 2 - 1)
        def _():
            start_copy(c0 + 2, lb0, gb0, sl0, sg0)

        wait_copy(c0 + 1, lb1, gb1, sl1, sg1)
        process(lb1, gb1)
        return carry

    lax.fori_loop(0, NCH // 2, pair_body, 0)
    pltpu.sync_copy(hist, out_hbm.at[pl.ds(w * ROW, ROW)])


_sc_hist = pl.kernel(
    _sc_hist_body,
    mesh=plsc.VectorSubcoreMesh(core_axis_name="c", subcore_axis_name="s",
                                num_cores=2, num_subcores=16),
    out_type=jax.ShapeDtypeStruct((NW * ROW,), jnp.float32),
    scratch_types=[
        pltpu.VMEM((ROW,), jnp.float32),
        pltpu.VMEM((RB, NC), jnp.float32),
        pltpu.VMEM((RB, NC), jnp.float32),
        pltpu.VMEM((RB, NC), jnp.int32),
        pltpu.VMEM((RB, NC), jnp.int32),
        pltpu.SemaphoreType.DMA,
        pltpu.SemaphoreType.DMA,
        pltpu.SemaphoreType.DMA,
        pltpu.SemaphoreType.DMA,
    ],
    compiler_params=pltpu.CompilerParams(needs_layout_passes=False,
                                         use_tc_tiling_on_sc=True),
)


def _cumsum_lanes(x):
    s = 1
    while s < NB:
        pad = jnp.zeros((B, s), jnp.float32)
        x = x + jnp.concatenate([pad, x[:, :NB - s]], axis=1)
        s *= 2
    return x


def _tc_reduce_body(n0_ref, n1_ref, s0_ref, s1_ref, out_ref):
    n0 = n0_ref[:B, :] + n0_ref[B:, :]
    n1 = n1_ref[:B, :] + n1_ref[B:, :]
    s0 = s0_ref[:B, :] + s0_ref[B:, :]
    s1 = s1_ref[:B, :] + s1_ref[B:, :]

    S1 = _cumsum_lanes(n1)
    S0 = _cumsum_lanes(n0)
    G = S1[:, NB - 1:]
    G2 = float(P) - G
    C_above = G - S1
    Z_above = S0[:, NB - 1:] - S0

    avg0 = jnp.where(n0 > 0, s0 / jnp.maximum(n0, 1.0), 0.0)
    avg1 = jnp.where(n1 > 0, s1 / jnp.maximum(n1, 1.0), 0.0)

    UA = G + Z_above
    UAn = UA + n0
    invA = jnp.where(UA > 0, 1.0 / jnp.maximum(UA, 1.0), 0.0)
    invAn = jnp.where(UAn > 0, 1.0 / jnp.maximum(UAn, 1.0), 0.0)
    lossA = jnp.sum(s1 * invA + avg0 * (G - C_above - n1) * (invA - invAn),
                    axis=1, keepdims=True)

    UB = G2 + C_above
    UBn = UB + n1
    invB = jnp.where(UB > 0, 1.0 / jnp.maximum(UB, 1.0), 0.0)
    invBn = jnp.where(UBn > 0, 1.0 / jnp.maximum(UBn, 1.0), 0.0)
    lossB = jnp.sum(s0 * invB + avg1 * (G2 - Z_above - n0) * (invB - invBn),
                    axis=1, keepdims=True)

    kk = lax.broadcasted_iota(jnp.int32, (B, NB), 1).astype(jnp.float32)
    est = jnp.max(jnp.where(n0 + n1 > 0, (kk + 1.0) * (T / NB), 0.0),
                  axis=1, keepdims=True)
    lossA = jnp.where(G > 0, lossA, est)
    lossB = jnp.where(G2 > 0, lossB, est)

    tot = jnp.sum((lossA + lossB) * 0.5, axis=0, keepdims=True) / float(B)
    out_ref[...] = tot


def kernel(logits, labels):
    flat = _sc_hist(logits.reshape(NR, NC), labels.reshape(NR, NC))
    hist = flat.reshape(NW, ROW)
    n0 = hist[:, 0:NB]
    n1 = hist[:, NB:2 * NB]
    s0 = hist[:, 2 * NB:3 * NB]
    s1 = hist[:, 3 * NB:4 * NB]
    loss = pl.pallas_call(
        _tc_reduce_body,
        out_shape=jax.ShapeDtypeStruct((1, 1), jnp.float32),
    )(n0, n1, s0, s1)
    return loss[0, 0]


# speedup vs baseline: 1.1371x; 1.1371x over previous
import jax
import jax.numpy as jnp
from jax import lax
from jax.experimental import pallas as pl
from jax.experimental.pallas import tpu as pltpu
from jax.experimental.pallas import tpu_sc as plsc

B = 16
P = 512 * 512
NB = 8192
T = 8.0
NW = 32
NR = 8192
NC = 512
RW = NR // NW
RB = 32
NCH = RW // RB
CHW = RB * NC
ROW = 4 * NB + 16
SCALE = NB / T


def _sc_hist_body(logits_hbm, labels_hbm, out_hbm, hist, lb0, lb1, gb0, gb1,
                  sl0, sg0, sl1, sg1):
    w = lax.axis_index("s") * 2 + lax.axis_index("c")
    img = w % B
    half = w // B
    r0 = img * (NR // B) + half * RW

    @plsc.parallel_loop(0, ROW, 16, unroll=8)
    def _(i):
        hist[pl.ds(i, 16)] = jnp.zeros((16,), jnp.float32)

    def start_copy(ci, lbuf, gbuf, seml, semg):
        rows = pl.ds(r0 + ci * RB, RB)
        pltpu.make_async_copy(logits_hbm.at[rows], lbuf, seml).start()
        pltpu.make_async_copy(labels_hbm.at[rows], gbuf, semg).start()

    def wait_copy(ci, lbuf, gbuf, seml, semg):
        rows = pl.ds(r0 + ci * RB, RB)
        pltpu.make_async_copy(logits_hbm.at[rows], lbuf, seml).wait()
        pltpu.make_async_copy(labels_hbm.at[rows], gbuf, semg).wait()

    def process(lbuf, gbuf, acc):
        def body(off, acc):
            r = lax.shift_right_logical(off, 9)
            c = lax.bitwise_and(off, NC - 1)
            lv = lbuf[r, pl.ds(c, 16)]
            gv = gbuf[r, pl.ds(c, 16)]
            gf = gv.astype(jnp.float32)
            e = 1.0 - lv * (2.0 * gf - 1.0)
            m = e > 0.0
            ki = jnp.clip((e * SCALE).astype(jnp.int32), 0, NB - 1)
            idx = lax.shift_left(gv, 13) + ki
            plsc.addupdate_scatter(hist, [idx], jnp.ones((16,), jnp.float32),
                                   mask=m)
            plsc.addupdate_scatter(hist, [idx + 2 * NB], e, mask=m)
            return acc + gf

        return plsc.parallel_loop(0, CHW, 16, unroll=8, carry=acc)(body)

    start_copy(0, lb0, gb0, sl0, sg0)

    def pair_body(ii, acc):
        c0 = 2 * ii
        start_copy(c0 + 1, lb1, gb1, sl1, sg1)
        wait_copy(c0, lb0, gb0, sl0, sg0)
        acc = process(lb0, gb0, acc)

        @pl.when(ii < NCH // 2 - 1)
        def _():
            start_copy(c0 + 2, lb0, gb0, sl0, sg0)

        wait_copy(c0 + 1, lb1, gb1, sl1, sg1)
        return process(lb1, gb1, acc)

    acc = lax.fori_loop(0, NCH // 2, pair_body, jnp.zeros((16,), jnp.float32))
    hist[pl.ds(4 * NB, 16)] = acc
    pltpu.sync_copy(hist, out_hbm.at[pl.ds(w * ROW, ROW)])


_sc_hist = pl.kernel(
    _sc_hist_body,
    mesh=plsc.VectorSubcoreMesh(core_axis_name="c", subcore_axis_name="s",
                                num_cores=2, num_subcores=16),
    out_type=jax.ShapeDtypeStruct((NW * ROW,), jnp.float32),
    scratch_types=[
        pltpu.VMEM((ROW,), jnp.float32),
        pltpu.VMEM((RB, NC), jnp.float32),
        pltpu.VMEM((RB, NC), jnp.float32),
        pltpu.VMEM((RB, NC), jnp.int32),
        pltpu.VMEM((RB, NC), jnp.int32),
        pltpu.SemaphoreType.DMA,
        pltpu.SemaphoreType.DMA,
        pltpu.SemaphoreType.DMA,
        pltpu.SemaphoreType.DMA,
    ],
    compiler_params=pltpu.CompilerParams(needs_layout_passes=False,
                                         use_tc_tiling_on_sc=True),
)


def _cumsum_lanes(x):
    s = 1
    while s < NB:
        pad = jnp.zeros((B, s), jnp.float32)
        x = x + jnp.concatenate([pad, x[:, :NB - s]], axis=1)
        s *= 2
    return x


def _tc_reduce_body(n0_ref, n1_ref, s0_ref, s1_ref, ac_ref, out_ref):
    n0 = n0_ref[:B, :] + n0_ref[B:, :]
    n1 = n1_ref[:B, :] + n1_ref[B:, :]
    s0 = s0_ref[:B, :] + s0_ref[B:, :]
    s1 = s1_ref[:B, :] + s1_ref[B:, :]
    G = jnp.sum(ac_ref[:B, :] + ac_ref[B:, :], axis=1, keepdims=True)
    G2 = float(P) - G

    S1 = _cumsum_lanes(n1)
    S0 = _cumsum_lanes(n0)
    C_above = S1[:, NB - 1:] - S1
    Z_above = S0[:, NB - 1:] - S0

    avg0 = jnp.where(n0 > 0, s0 / jnp.maximum(n0, 1.0), 0.0)
    avg1 = jnp.where(n1 > 0, s1 / jnp.maximum(n1, 1.0), 0.0)

    UA = G + Z_above
    UAn = UA + n0
    invA = jnp.where(UA > 0, 1.0 / jnp.maximum(UA, 1.0), 0.0)
    invAn = jnp.where(UAn > 0, 1.0 / jnp.maximum(UAn, 1.0), 0.0)
    lossA = jnp.sum(s1 * invA + avg0 * (G - C_above - n1) * (invA - invAn),
                    axis=1, keepdims=True)

    UB = G2 + C_above
    UBn = UB + n1
    invB = jnp.where(UB > 0, 1.0 / jnp.maximum(UB, 1.0), 0.0)
    invBn = jnp.where(UBn > 0, 1.0 / jnp.maximum(UBn, 1.0), 0.0)
    lossB = jnp.sum(s0 * invB + avg1 * (G2 - Z_above - n0) * (invB - invBn),
                    axis=1, keepdims=True)

    kk = lax.broadcasted_iota(jnp.int32, (B, NB), 1).astype(jnp.float32)
    est = jnp.max(jnp.where(n0 + n1 > 0, (kk + 1.0) * (T / NB), 0.0),
                  axis=1, keepdims=True)
    lossA = jnp.where(G > 0, lossA, est)
    lossB = jnp.where(G2 > 0, lossB, est)

    tot = jnp.sum((lossA + lossB) * 0.5, axis=0, keepdims=True) / float(B)
    out_ref[...] = tot


def kernel(logits, labels):
    flat = _sc_hist(logits.reshape(NR, NC), labels.reshape(NR, NC))
    hist = flat.reshape(NW, ROW)
    n0 = hist[:, 0:NB]
    n1 = hist[:, NB:2 * NB]
    s0 = hist[:, 2 * NB:3 * NB]
    s1 = hist[:, 3 * NB:4 * NB]
    ac = hist[:, 4 * NB:]
    loss = pl.pallas_call(
        _tc_reduce_body,
        out_shape=jax.ShapeDtypeStruct((1, 1), jnp.float32),
    )(n0, n1, s0, s1, ac)
    return loss[0, 0]
